# baseline (speedup 1.0000x reference)
import functools

import jax
import jax.numpy as jnp
from jax import lax
from jax.experimental import pallas as pl
from jax.experimental.pallas import tpu as pltpu
from jax.experimental.pallas import tpu_sc as plsc

EMBED_DIM = 128
NUM_SYM = 100000
PAD = NUM_SYM
KMAX = 200
KSEL = 10
B_Q, B_S = 1024, 5
NROWS = 2 * B_Q + 2 * B_S
NB = 2304
NW = 32
CHUNK = 128


def _mm_t(a, b):
    return lax.dot_general(a, b, (((1,), (1,)), ((), ())),
                           preferred_element_type=jnp.float32)


P_W = NB // NW
HALF = KMAX // 2
PROWS = 56


def _sc_sims(table, ids3, selfids2):
    mesh = plsc.VectorSubcoreMesh(core_axis_name="c", subcore_axis_name="s")

    @functools.partial(
        pl.kernel, mesh=mesh,
        out_type=[
            jax.ShapeDtypeStruct((NB * PROWS, EMBED_DIM), jnp.float32),
            jax.ShapeDtypeStruct((NB, EMBED_DIM), jnp.float32),
        ],
        scratch_types=[
            pltpu.VMEM((2 * P_W, HALF), jnp.int32),
            pltpu.VMEM((P_W,), jnp.int32),
            pltpu.VMEM((P_W, EMBED_DIM), jnp.float32),
            pltpu.VMEM((KMAX, EMBED_DIM), jnp.float32),
            pltpu.VMEM((KMAX, EMBED_DIM), jnp.float32),
            pltpu.VMEM((KMAX, EMBED_DIM), jnp.float32),
            pltpu.VMEM((PROWS, EMBED_DIM), jnp.float32),
            pltpu.VMEM((PROWS, EMBED_DIM), jnp.float32),
            pltpu.VMEM((PROWS, EMBED_DIM), jnp.float32),
            pltpu.SemaphoreType.DMA,
            pltpu.SemaphoreType.DMA,
            pltpu.SemaphoreType.DMA,
            pltpu.SemaphoreType.DMA,
            pltpu.SemaphoreType.DMA,
            pltpu.SemaphoreType.DMA,
            pltpu.SemaphoreType.DMA,
        ],
    )
    def k(table_hbm, ids_hbm, sids_hbm, parts_hbm, selfs_hbm, ids_v, sidx_v,
          selfs_v, ent0, ent1, ent2, pb0, pb1, pb2, sem0, sem1, sem2,
          wsem0, wsem1, wsem2, sems):
        wid = lax.axis_index("s") * 2 + lax.axis_index("c")
        pltpu.sync_copy(ids_hbm.at[wid], ids_v)
        pltpu.sync_copy(sids_hbm.at[wid], sidx_v)
        pltpu.make_async_copy(table_hbm.at[sidx_v], selfs_v, sems).start()
        pltpu.make_async_copy(table_hbm.at[sidx_v], selfs_v, sems).wait()
        pltpu.sync_copy(selfs_v, selfs_hbm.at[pl.ds(wid * P_W, P_W)])
        ents = (ent0, ent1, ent2)
        sems2 = (sem0, sem1, sem2)
        pbs = (pb0, pb1, pb2)
        wsems = (wsem0, wsem1, wsem2)

        def poff(p):
            return (wid * P_W + p) * PROWS

        def start(p, b):
            for h in range(2):
                pltpu.make_async_copy(
                    table_hbm.at[ids_v.at[2 * p + h]],
                    ents[b].at[pl.ds(h * HALF, HALF)], sems2[b]).start()

        def finish(p, b):
            for h in range(2):
                pltpu.make_async_copy(
                    table_hbm.at[ids_v.at[2 * p + h]],
                    ents[b].at[pl.ds(h * HALF, HALF)], sems2[b]).wait()
            ent, pb = ents[b], pbs[b]

            @pl.when(p >= 3)
            def _():
                pltpu.make_async_copy(
                    pb, parts_hbm.at[pl.ds(poff(p - 3), PROWS)],
                    wsems[b]).wait()

            sv = [selfs_v[p, pl.ds(16 * kk, 16)] for kk in range(8)]

            def rows8(i, carry):
                for jj in range(8):
                    j = 8 * i + jj
                    es = [ent[j, pl.ds(16 * kk, 16)] for kk in range(8)]
                    pr = [es[kk] * sv[kk] for kk in range(8)]
                    sp = [es[kk] * es[kk] for kk in range(8)]
                    acc = ((pr[0] + pr[1]) + (pr[2] + pr[3])) + (
                        (pr[4] + pr[5]) + (pr[6] + pr[7]))
                    sq = ((sp[0] + sp[1]) + (sp[2] + sp[3])) + (
                        (sp[4] + sp[5]) + (sp[6] + sp[7]))
                    pb[i, pl.ds(16 * jj, 16)] = acc
                    pb[25 + i, pl.ds(16 * jj, 16)] = sq
                return carry

            lax.fori_loop(0, 25, rows8, 0)
            pltpu.make_async_copy(
                pb, parts_hbm.at[pl.ds(poff(p), PROWS)], wsems[b]).start()

        start(0, 0)
        start(1, 1)

        def body(i, carry):
            for u in range(3):
                p = 3 * i + u

                @pl.when(p + 2 < P_W)
                def _():
                    start(p + 2, (u + 2) % 3)

                finish(p, u)
            return carry

        lax.fori_loop(0, P_W // 3, body, 0)
        for u in range(3):
            pltpu.make_async_copy(
                pbs[u], parts_hbm.at[pl.ds(poff(P_W - 3 + u), PROWS)],
                wsems[u]).wait()

    return k(table, ids3, selfids2)


def _sc_gather(table, ids_3d, nch, chunk):
    n_out = NW * nch * chunk
    per_w = nch * chunk
    mesh = plsc.VectorSubcoreMesh(core_axis_name="c", subcore_axis_name="s")

    @functools.partial(
        pl.kernel, mesh=mesh,
        out_type=jax.ShapeDtypeStruct((n_out, EMBED_DIM), jnp.float32),
        scratch_types=[
            pltpu.VMEM((1, nch * chunk), jnp.int32),
            pltpu.VMEM((chunk, EMBED_DIM), jnp.float32),
            pltpu.VMEM((chunk, EMBED_DIM), jnp.float32),
            pltpu.VMEM((chunk, EMBED_DIM), jnp.float32),
            pltpu.VMEM((chunk, EMBED_DIM), jnp.float32),
            pltpu.SemaphoreType.DMA,
            pltpu.SemaphoreType.DMA,
            pltpu.SemaphoreType.DMA,
            pltpu.SemaphoreType.DMA,
            pltpu.SemaphoreType.DMA,
            pltpu.SemaphoreType.DMA,
            pltpu.SemaphoreType.DMA,
            pltpu.SemaphoreType.DMA,
        ],
    )
    def k(table_hbm, ids_hbm, out_hbm, idx_v, b0, b1, b2, b3, g0, g1, g2,
          g3, w0, w1, w2, w3):
        wid = lax.axis_index("s") * 2 + lax.axis_index("c")
        pltpu.sync_copy(ids_hbm.at[wid], idx_v)
        bufs = (b0, b1, b2, b3)
        gsems = (g0, g1, g2, g3)
        wsems = (w0, w1, w2, w3)

        def gcopy(ci):
            return pltpu.make_async_copy(
                table_hbm.at[idx_v.at[0, pl.ds(ci * chunk, chunk)]],
                bufs[ci % 4], gsems[ci % 4])

        def wcopy(ci):
            off = wid * per_w + ci * chunk
            return pltpu.make_async_copy(
                bufs[ci % 4], out_hbm.at[pl.ds(off, chunk)], wsems[ci % 4])

        for ci in range(min(2, nch)):
            gcopy(ci).start()
        for ci in range(nch):
            if ci >= 2:
                wcopy(ci - 2).wait()
            if ci + 2 < nch:
                gcopy(ci + 2).start()
            gcopy(ci).wait()
            wcopy(ci).start()
        for ci in range(max(0, nch - 2), nch):
            wcopy(ci).wait()

    return k(table, ids_3d)


def _tc_groupsum(parts, blk):
    grid = NB // blk

    def body(parts_ref, dot_ref, esq_ref):
        pa = parts_ref[...].reshape(blk, PROWS, EMBED_DIM)
        m8 = (lax.broadcasted_iota(jnp.int32, (EMBED_DIM, 8), 0) // 16
              == lax.broadcasted_iota(jnp.int32, (EMBED_DIM, 8), 1)
              ).astype(jnp.float32)
        dims = (((1,), (0,)), ((), ()))

        def groups(base):
            cols = [lax.dot_general(
                pa[:, base + r, :], m8, dims,
                precision=lax.Precision.HIGHEST,
                preferred_element_type=jnp.float32) for r in range(25)]
            return jnp.concatenate(cols, axis=1)

        dot_ref[...] = groups(0)
        esq_ref[...] = groups(25)

    return pl.pallas_call(
        body,
        grid=(grid,),
        in_specs=[pl.BlockSpec((blk * PROWS, EMBED_DIM), lambda g: (g, 0))],
        out_specs=[
            pl.BlockSpec((blk, KMAX), lambda g: (g, 0)),
            pl.BlockSpec((blk, KMAX), lambda g: (g, 0)),
        ],
        out_shape=[
            jax.ShapeDtypeStruct((NB, KMAX), jnp.float32),
            jax.ShapeDtypeStruct((NB, KMAX), jnp.float32),
        ],
    )(parts)


def _tc_simtopk(dot2d, esq2d, selfs, rel_ids, ent_ids, blk):
    grid = NB // blk

    def body(dot_ref, esq_ref, self_ref, rel_ref, eid_ref, relo_ref,
             ento_ref):
        dot = dot_ref[...]
        esq = esq_ref[...]
        selfr = self_ref[...]
        rels = rel_ref[...]
        eids = eid_ref[...]

        ssq = jnp.sum(selfr * selfr, axis=-1, keepdims=True)
        sim = dot * lax.rsqrt(jnp.maximum(esq * ssq, 1e-24))
        sim = sim - jnp.where(rels == PAD, 1e9, 0.0).astype(jnp.float32)

        iota = lax.broadcasted_iota(jnp.int32, (blk, KMAX), 1)
        relf = rels.astype(jnp.float32)
        entf = eids.astype(jnp.float32)
        rel_cols, ent_cols = [], []
        for _ in range(KSEL):
            m = jnp.max(sim, axis=1, keepdims=True)
            ismax = sim == m
            idxsel = jnp.min(jnp.where(ismax, iota, KMAX + 1), axis=1,
                             keepdims=True)
            chosen = iota == idxsel
            rel_cols.append(jnp.sum(jnp.where(chosen, relf, 0.0), axis=1,
                                    keepdims=True))
            ent_cols.append(jnp.sum(jnp.where(chosen, entf, 0.0), axis=1,
                                    keepdims=True))
            sim = jnp.where(chosen, -1e38, sim)
        relo_ref[...] = jnp.concatenate(rel_cols, axis=1).astype(jnp.int32)
        ento_ref[...] = jnp.concatenate(ent_cols, axis=1).astype(jnp.int32)

    return pl.pallas_call(
        body,
        grid=(grid,),
        in_specs=[
            pl.BlockSpec((blk, KMAX), lambda g: (g, 0)),
            pl.BlockSpec((blk, KMAX), lambda g: (g, 0)),
            pl.BlockSpec((blk, EMBED_DIM), lambda g: (g, 0)),
            pl.BlockSpec((blk, KMAX), lambda g: (g, 0)),
            pl.BlockSpec((blk, KMAX), lambda g: (g, 0)),
        ],
        out_specs=[
            pl.BlockSpec((blk, KSEL), lambda g: (g, 0)),
            pl.BlockSpec((blk, KSEL), lambda g: (g, 0)),
        ],
        out_shape=[
            jax.ShapeDtypeStruct((NB, KSEL), jnp.int32),
            jax.ShapeDtypeStruct((NB, KSEL), jnp.int32),
        ],
    )(dot2d, esq2d, selfs, rel_ids, ent_ids)


def _tc_neighbor(rows_c, selfs, gcn_w, gcn_wb, gcn_b, gate_w, gate_wb,
                 gate_b, blk):
    grid = NB // blk

    def body(pair_ref, self_ref, gw_ref, gwb_ref, gb_ref, gatew_ref,
             gatewb_ref, gateb_ref, out_ref):
        pairs = pair_ref[...].reshape(blk, KSEL, 2 * EMBED_DIM)
        selfr = self_ref[...]
        proj = lax.dot_general(pairs, gw_ref[...],
                               (((2,), (1,)), ((), ())),
                               preferred_element_type=jnp.float32)
        proj = proj + (gwb_ref[...] + gb_ref[...])[None, None, :]
        proj = jnp.where(proj >= 0, proj, 0.01 * proj)
        agg = jnp.sum(proj, axis=1) / (float(KSEL) + 1e-9)
        lin = jnp.sum(agg * gatew_ref[...], axis=1, keepdims=True)
        gate = jax.nn.sigmoid(lin + (gatewb_ref[0] + gateb_ref[0]))
        final = gate * agg + (1.0 - gate) * selfr
        out_ref[...] = jnp.tanh(final)

    return pl.pallas_call(
        body,
        grid=(grid,),
        in_specs=[
            pl.BlockSpec((blk * 2 * KSEL, EMBED_DIM), lambda g: (g, 0)),
            pl.BlockSpec((blk, EMBED_DIM), lambda g: (g, 0)),
            pl.BlockSpec((EMBED_DIM, 2 * EMBED_DIM), lambda g: (0, 0)),
            pl.BlockSpec((EMBED_DIM,), lambda g: (0,)),
            pl.BlockSpec((EMBED_DIM,), lambda g: (0,)),
            pl.BlockSpec((1, EMBED_DIM), lambda g: (0, 0)),
            pl.BlockSpec((1,), lambda g: (0,)),
            pl.BlockSpec((1,), lambda g: (0,)),
        ],
        out_specs=pl.BlockSpec((blk, EMBED_DIM), lambda g: (g, 0)),
        out_shape=jax.ShapeDtypeStruct((NB, EMBED_DIM), jnp.float32),
    )(rows_c, selfs, gcn_w, gcn_wb, gcn_b, gate_w, gate_wb, gate_b)


def _tc_tail(query_vec, support_vec, p1w, p1b, p2w, p2b, lng, lnb, wih, whh,
             bih, bhh, blk):
    d_model = 2 * EMBED_DIM
    grid = B_Q // blk

    def enc(x, p1w, p1b, p2w, p2b, lng, lnb):
        out = jax.nn.relu(_mm_t(x, p1w) + p1b[None, :])
        out = _mm_t(out, p2w) + p2b[None, :]
        y = out + x
        mu = jnp.mean(y, axis=-1, keepdims=True)
        var = jnp.mean((y - mu) ** 2, axis=-1, keepdims=True)
        return lng[None, :] * (y - mu) * lax.rsqrt(var + 1e-6) + lnb[None, :]

    def body(q_ref, sv_ref, p1w_ref, p1b_ref, p2w_ref, p2b_ref, lng_ref,
             lnb_ref, wih_ref, whh_ref, bih_ref, bhh_ref, out_ref):
        p1w, p1b = p1w_ref[...], p1b_ref[...]
        p2w, p2b = p2w_ref[...], p2b_ref[...]
        lng, lnb = lng_ref[...], lnb_ref[...]
        wih, whh = wih_ref[...], whh_ref[...]
        bias = (bih_ref[...] + bhh_ref[...])[None, :]

        sg = jnp.mean(enc(sv_ref[...], p1w, p1b, p2w, p2b, lng, lnb),
                      axis=0, keepdims=True)
        qe = enc(q_ref[...], p1w, p1b, p2w, p2b, lng, lnb)

        qc = _mm_t(qe, wih) + bias
        whh_l = whh[:, :d_model]
        whh_r = whh[:, d_model:]
        rcon = _mm_t(sg, whh_r)

        hid = 2 * d_model
        c = jnp.zeros((blk, hid), jnp.float32)
        h = qe
        for step in range(4):
            if step == 0:
                gates = qc
            else:
                gates = qc + _mm_t(h, whh_l) + rcon
            gi = gates[:, 0 * hid:1 * hid]
            gf = gates[:, 1 * hid:2 * hid]
            gg = gates[:, 2 * hid:3 * hid]
            go = gates[:, 3 * hid:4 * hid]
            c = jax.nn.sigmoid(gf) * c + jax.nn.sigmoid(gi) * jnp.tanh(gg)
            h_r = jax.nn.sigmoid(go) * jnp.tanh(c)
            h = qe + h_r[:, :d_model]
        out_ref[...] = jnp.sum(h * sg, axis=1)

    return pl.pallas_call(
        body,
        grid=(grid,),
        in_specs=[
            pl.BlockSpec((blk, d_model), lambda g: (g, 0)),
            pl.BlockSpec((B_S, d_model), lambda g: (0, 0)),
            pl.BlockSpec((2 * d_model, d_model), lambda g: (0, 0)),
            pl.BlockSpec((2 * d_model,), lambda g: (0,)),
            pl.BlockSpec((d_model, 2 * d_model), lambda g: (0, 0)),
            pl.BlockSpec((d_model,), lambda g: (0,)),
            pl.BlockSpec((d_model,), lambda g: (0,)),
            pl.BlockSpec((d_model,), lambda g: (0,)),
            pl.BlockSpec((8 * d_model, d_model), lambda g: (0, 0)),
            pl.BlockSpec((8 * d_model, 2 * d_model), lambda g: (0, 0)),
            pl.BlockSpec((8 * d_model,), lambda g: (0,)),
            pl.BlockSpec((8 * d_model,), lambda g: (0,)),
        ],
        out_specs=pl.BlockSpec((blk,), lambda g: (g,)),
        out_shape=jax.ShapeDtypeStruct((B_Q,), jnp.float32),
    )(query_vec, support_vec, p1w, p1b, p2w, p2b, lng, lnb, wih, whh, bih,
      bhh)


def kernel(symbol_emb, gcn_w, gcn_wb, gcn_b, gate_w, gate_wb, gate_b, p1w,
           p1b, p2w, p2b, lng, lnb, wih, whh, bih, bhh, query, support, q_l1,
           q_e2, q_deg_l, q_r1, q_e5, q_deg_r, s_l1, s_e2, s_deg_l, s_r1,
           s_e5, s_deg_r):
    npad = NB - NROWS
    conn = jnp.concatenate([q_l1, q_r1, s_l1, s_r1], axis=0)
    conn_fill = (jnp.arange(npad * KMAX * 2, dtype=jnp.int32) % NUM_SYM
                 ).reshape(npad, KMAX, 2)
    conn = jnp.concatenate([conn, conn_fill], axis=0)
    selves = jnp.concatenate([query[:, 0], query[:, 1], support[:, 0],
                              support[:, 1],
                              jnp.arange(npad, dtype=jnp.int32) % NUM_SYM])
    rel_ids = conn[:, :, 0]
    ent_ids = conn[:, :, 1]

    ids3 = ent_ids.reshape(NW, 2 * P_W, HALF)
    selfids2 = selves.reshape(NW, P_W)
    parts, selfrows = _sc_sims(symbol_emb, ids3, selfids2)

    dot2d, esq2d = _tc_groupsum(parts, blk=128)
    rel_sel, ent_sel = _tc_simtopk(dot2d, esq2d, selfrows,
                                   rel_ids, ent_ids, blk=128)

    ids_c = jnp.stack([rel_sel, ent_sel], axis=-1).reshape(-1)
    pad_c = NW * 12 * 128 - ids_c.shape[0]
    ids_c = jnp.concatenate(
        [ids_c, jnp.arange(pad_c, dtype=jnp.int32) % NUM_SYM]
    ).reshape(NW, 1, 12 * 128)
    rows_c = _sc_gather(symbol_emb, ids_c, nch=12, chunk=128)

    nbout = _tc_neighbor(rows_c, selfrows, gcn_w, gcn_wb, gcn_b, gate_w,
                         gate_wb, gate_b, blk=128)

    query_vec = jnp.concatenate([nbout[:B_Q], nbout[B_Q:2 * B_Q]], axis=1)
    support_vec = jnp.concatenate(
        [nbout[2 * B_Q:2 * B_Q + B_S], nbout[2 * B_Q + B_S:NROWS]], axis=1)

    return _tc_tail(query_vec, support_vec, p1w, p1b, p2w, p2b, lng, lnb,
                    wih, whh, bih, bhh, blk=256)


# speedup vs baseline: 1.1343x; 1.1343x over previous
import functools

import jax
import jax.numpy as jnp
from jax import lax
from jax.experimental import pallas as pl
from jax.experimental.pallas import tpu as pltpu
from jax.experimental.pallas import tpu_sc as plsc

EMBED_DIM = 128
NUM_SYM = 100000
PAD = NUM_SYM
KMAX = 200
KSEL = 10
B_Q, B_S = 1024, 5
NROWS = 2 * B_Q + 2 * B_S
NB = 2304
NW = 32
CHUNK = 128


def _mm_t(a, b):
    return lax.dot_general(a, b, (((1,), (1,)), ((), ())),
                           preferred_element_type=jnp.float32)


P_W = NB // NW
HALF = KMAX // 2
PROWS = 56


def _sc_sims(table, ids3, selfids2, nb):
    pw = nb // NW
    mesh = plsc.VectorSubcoreMesh(core_axis_name="c", subcore_axis_name="s")

    @functools.partial(
        pl.kernel, mesh=mesh,
        out_type=[
            jax.ShapeDtypeStruct((nb * PROWS, EMBED_DIM), jnp.float32),
            jax.ShapeDtypeStruct((nb, EMBED_DIM), jnp.float32),
        ],
        scratch_types=[
            pltpu.VMEM((2 * pw, HALF), jnp.int32),
            pltpu.VMEM((pw,), jnp.int32),
            pltpu.VMEM((pw, EMBED_DIM), jnp.float32),
            pltpu.VMEM((KMAX, EMBED_DIM), jnp.float32),
            pltpu.VMEM((KMAX, EMBED_DIM), jnp.float32),
            pltpu.VMEM((KMAX, EMBED_DIM), jnp.float32),
            pltpu.VMEM((PROWS, EMBED_DIM), jnp.float32),
            pltpu.VMEM((PROWS, EMBED_DIM), jnp.float32),
            pltpu.VMEM((PROWS, EMBED_DIM), jnp.float32),
            pltpu.SemaphoreType.DMA,
            pltpu.SemaphoreType.DMA,
            pltpu.SemaphoreType.DMA,
            pltpu.SemaphoreType.DMA,
            pltpu.SemaphoreType.DMA,
            pltpu.SemaphoreType.DMA,
            pltpu.SemaphoreType.DMA,
        ],
    )
    def k(table_hbm, ids_hbm, sids_hbm, parts_hbm, selfs_hbm, ids_v, sidx_v,
          selfs_v, ent0, ent1, ent2, pb0, pb1, pb2, sem0, sem1, sem2,
          wsem0, wsem1, wsem2, sems):
        wid = lax.axis_index("s") * 2 + lax.axis_index("c")
        pltpu.sync_copy(ids_hbm.at[wid], ids_v)
        pltpu.sync_copy(sids_hbm.at[wid], sidx_v)
        pltpu.make_async_copy(table_hbm.at[sidx_v], selfs_v, sems).start()
        pltpu.make_async_copy(table_hbm.at[sidx_v], selfs_v, sems).wait()
        pltpu.sync_copy(selfs_v, selfs_hbm.at[pl.ds(wid * pw, pw)])
        ents = (ent0, ent1, ent2)
        sems2 = (sem0, sem1, sem2)
        pbs = (pb0, pb1, pb2)
        wsems = (wsem0, wsem1, wsem2)

        def poff(p):
            return (wid * pw + p) * PROWS

        def start(p, b):
            for h in range(2):
                pltpu.make_async_copy(
                    table_hbm.at[ids_v.at[2 * p + h]],
                    ents[b].at[pl.ds(h * HALF, HALF)], sems2[b]).start()

        def finish(p, b):
            for h in range(2):
                pltpu.make_async_copy(
                    table_hbm.at[ids_v.at[2 * p + h]],
                    ents[b].at[pl.ds(h * HALF, HALF)], sems2[b]).wait()
            ent, pb = ents[b], pbs[b]

            @pl.when(p >= 3)
            def _():
                pltpu.make_async_copy(
                    pb, parts_hbm.at[pl.ds(poff(p - 3), PROWS)],
                    wsems[b]).wait()

            sv = [selfs_v[p, pl.ds(16 * kk, 16)] for kk in range(8)]

            def rows8(i, carry):
                for jj in range(8):
                    j = 8 * i + jj
                    es = [ent[j, pl.ds(16 * kk, 16)] for kk in range(8)]
                    pr = [es[kk] * sv[kk] for kk in range(8)]
                    sp = [es[kk] * es[kk] for kk in range(8)]
                    acc = ((pr[0] + pr[1]) + (pr[2] + pr[3])) + (
                        (pr[4] + pr[5]) + (pr[6] + pr[7]))
                    sq = ((sp[0] + sp[1]) + (sp[2] + sp[3])) + (
                        (sp[4] + sp[5]) + (sp[6] + sp[7]))
                    pb[i, pl.ds(16 * jj, 16)] = acc
                    pb[25 + i, pl.ds(16 * jj, 16)] = sq
                return carry

            lax.fori_loop(0, 25, rows8, 0)
            pltpu.make_async_copy(
                pb, parts_hbm.at[pl.ds(poff(p), PROWS)], wsems[b]).start()

        start(0, 0)
        start(1, 1)

        def body(i, carry):
            for u in range(3):
                p = 3 * i + u

                @pl.when(p + 2 < pw)
                def _():
                    start(p + 2, (u + 2) % 3)

                finish(p, u)
            return carry

        lax.fori_loop(0, pw // 3, body, 0)
        for u in range(3):
            pltpu.make_async_copy(
                pbs[u], parts_hbm.at[pl.ds(poff(pw - 3 + u), PROWS)],
                wsems[u]).wait()

    return k(table, ids3, selfids2)


def _sc_gather(table, ids_3d, nch, chunk):
    n_out = NW * nch * chunk
    per_w = nch * chunk
    mesh = plsc.VectorSubcoreMesh(core_axis_name="c", subcore_axis_name="s")

    @functools.partial(
        pl.kernel, mesh=mesh,
        out_type=jax.ShapeDtypeStruct((n_out, EMBED_DIM), jnp.float32),
        scratch_types=[
            pltpu.VMEM((1, nch * chunk), jnp.int32),
            pltpu.VMEM((chunk, EMBED_DIM), jnp.float32),
            pltpu.VMEM((chunk, EMBED_DIM), jnp.float32),
            pltpu.VMEM((chunk, EMBED_DIM), jnp.float32),
            pltpu.VMEM((chunk, EMBED_DIM), jnp.float32),
            pltpu.SemaphoreType.DMA,
            pltpu.SemaphoreType.DMA,
            pltpu.SemaphoreType.DMA,
            pltpu.SemaphoreType.DMA,
            pltpu.SemaphoreType.DMA,
            pltpu.SemaphoreType.DMA,
            pltpu.SemaphoreType.DMA,
            pltpu.SemaphoreType.DMA,
        ],
    )
    def k(table_hbm, ids_hbm, out_hbm, idx_v, b0, b1, b2, b3, g0, g1, g2,
          g3, w0, w1, w2, w3):
        wid = lax.axis_index("s") * 2 + lax.axis_index("c")
        pltpu.sync_copy(ids_hbm.at[wid], idx_v)
        bufs = (b0, b1, b2, b3)
        gsems = (g0, g1, g2, g3)
        wsems = (w0, w1, w2, w3)

        def gcopy(ci):
            return pltpu.make_async_copy(
                table_hbm.at[idx_v.at[0, pl.ds(ci * chunk, chunk)]],
                bufs[ci % 4], gsems[ci % 4])

        def wcopy(ci):
            off = wid * per_w + ci * chunk
            return pltpu.make_async_copy(
                bufs[ci % 4], out_hbm.at[pl.ds(off, chunk)], wsems[ci % 4])

        for ci in range(min(2, nch)):
            gcopy(ci).start()
        for ci in range(nch):
            if ci >= 2:
                wcopy(ci - 2).wait()
            if ci + 2 < nch:
                gcopy(ci + 2).start()
            gcopy(ci).wait()
            wcopy(ci).start()
        for ci in range(max(0, nch - 2), nch):
            wcopy(ci).wait()

    return k(table, ids_3d)


def _tc_groupsum(parts, blk, nb):
    grid = nb // blk

    def body(parts_ref, dot_ref, esq_ref):
        pa = parts_ref[...].reshape(blk, PROWS, EMBED_DIM)
        m8 = (lax.broadcasted_iota(jnp.int32, (EMBED_DIM, 8), 0) // 16
              == lax.broadcasted_iota(jnp.int32, (EMBED_DIM, 8), 1)
              ).astype(jnp.float32)
        dims = (((1,), (0,)), ((), ()))

        def groups(base):
            cols = [lax.dot_general(
                pa[:, base + r, :], m8, dims,
                precision=lax.Precision.HIGHEST,
                preferred_element_type=jnp.float32) for r in range(25)]
            return jnp.concatenate(cols, axis=1)

        dot_ref[...] = groups(0)
        esq_ref[...] = groups(25)

    return pl.pallas_call(
        body,
        grid=(grid,),
        in_specs=[pl.BlockSpec((blk * PROWS, EMBED_DIM), lambda g: (g, 0))],
        out_specs=[
            pl.BlockSpec((blk, KMAX), lambda g: (g, 0)),
            pl.BlockSpec((blk, KMAX), lambda g: (g, 0)),
        ],
        out_shape=[
            jax.ShapeDtypeStruct((nb, KMAX), jnp.float32),
            jax.ShapeDtypeStruct((nb, KMAX), jnp.float32),
        ],
    )(parts)


def _tc_simtopk(dot2d, esq2d, selfs, rel_ids, ent_ids, blk, nb):
    grid = nb // blk

    def body(dot_ref, esq_ref, self_ref, rel_ref, eid_ref, relo_ref,
             ento_ref):
        dot = dot_ref[...]
        esq = esq_ref[...]
        selfr = self_ref[...]
        rels = rel_ref[...]
        eids = eid_ref[...]

        ssq = jnp.sum(selfr * selfr, axis=-1, keepdims=True)
        sim = dot * lax.rsqrt(jnp.maximum(esq * ssq, 1e-24))
        sim = sim - jnp.where(rels == PAD, 1e9, 0.0).astype(jnp.float32)

        iota = lax.broadcasted_iota(jnp.int32, (blk, KMAX), 1)
        relf = rels.astype(jnp.float32)
        entf = eids.astype(jnp.float32)
        rel_cols, ent_cols = [], []
        for _ in range(KSEL):
            m = jnp.max(sim, axis=1, keepdims=True)
            ismax = sim == m
            idxsel = jnp.min(jnp.where(ismax, iota, KMAX + 1), axis=1,
                             keepdims=True)
            chosen = iota == idxsel
            rel_cols.append(jnp.sum(jnp.where(chosen, relf, 0.0), axis=1,
                                    keepdims=True))
            ent_cols.append(jnp.sum(jnp.where(chosen, entf, 0.0), axis=1,
                                    keepdims=True))
            sim = jnp.where(chosen, -1e38, sim)
        relo_ref[...] = jnp.concatenate(rel_cols, axis=1).astype(jnp.int32)
        ento_ref[...] = jnp.concatenate(ent_cols, axis=1).astype(jnp.int32)

    return pl.pallas_call(
        body,
        grid=(grid,),
        in_specs=[
            pl.BlockSpec((blk, KMAX), lambda g: (g, 0)),
            pl.BlockSpec((blk, KMAX), lambda g: (g, 0)),
            pl.BlockSpec((blk, EMBED_DIM), lambda g: (g, 0)),
            pl.BlockSpec((blk, KMAX), lambda g: (g, 0)),
            pl.BlockSpec((blk, KMAX), lambda g: (g, 0)),
        ],
        out_specs=[
            pl.BlockSpec((blk, KSEL), lambda g: (g, 0)),
            pl.BlockSpec((blk, KSEL), lambda g: (g, 0)),
        ],
        out_shape=[
            jax.ShapeDtypeStruct((nb, KSEL), jnp.int32),
            jax.ShapeDtypeStruct((nb, KSEL), jnp.int32),
        ],
    )(dot2d, esq2d, selfs, rel_ids, ent_ids)


def _tc_neighbor(rows_c, selfs, gcn_w, gcn_wb, gcn_b, gate_w, gate_wb,
                 gate_b, blk, nb):
    grid = nb // blk

    def body(pair_ref, self_ref, gw_ref, gwb_ref, gb_ref, gatew_ref,
             gatewb_ref, gateb_ref, out_ref):
        pairs = pair_ref[...].reshape(blk, KSEL, 2 * EMBED_DIM)
        selfr = self_ref[...]
        proj = lax.dot_general(pairs, gw_ref[...],
                               (((2,), (1,)), ((), ())),
                               preferred_element_type=jnp.float32)
        proj = proj + (gwb_ref[...] + gb_ref[...])[None, None, :]
        proj = jnp.where(proj >= 0, proj, 0.01 * proj)
        agg = jnp.sum(proj, axis=1) / (float(KSEL) + 1e-9)
        lin = jnp.sum(agg * gatew_ref[...], axis=1, keepdims=True)
        gate = jax.nn.sigmoid(lin + (gatewb_ref[0] + gateb_ref[0]))
        final = gate * agg + (1.0 - gate) * selfr
        out_ref[...] = jnp.tanh(final)

    return pl.pallas_call(
        body,
        grid=(grid,),
        in_specs=[
            pl.BlockSpec((blk * 2 * KSEL, EMBED_DIM), lambda g: (g, 0)),
            pl.BlockSpec((blk, EMBED_DIM), lambda g: (g, 0)),
            pl.BlockSpec((EMBED_DIM, 2 * EMBED_DIM), lambda g: (0, 0)),
            pl.BlockSpec((EMBED_DIM,), lambda g: (0,)),
            pl.BlockSpec((EMBED_DIM,), lambda g: (0,)),
            pl.BlockSpec((1, EMBED_DIM), lambda g: (0, 0)),
            pl.BlockSpec((1,), lambda g: (0,)),
            pl.BlockSpec((1,), lambda g: (0,)),
        ],
        out_specs=pl.BlockSpec((blk, EMBED_DIM), lambda g: (g, 0)),
        out_shape=jax.ShapeDtypeStruct((nb, EMBED_DIM), jnp.float32),
    )(rows_c, selfs, gcn_w, gcn_wb, gcn_b, gate_w, gate_wb, gate_b)


def _tc_tail(query_vec, support_vec, p1w, p1b, p2w, p2b, lng, lnb, wih, whh,
             bih, bhh, blk):
    d_model = 2 * EMBED_DIM
    grid = B_Q // blk

    def enc(x, p1w, p1b, p2w, p2b, lng, lnb):
        out = jax.nn.relu(_mm_t(x, p1w) + p1b[None, :])
        out = _mm_t(out, p2w) + p2b[None, :]
        y = out + x
        mu = jnp.mean(y, axis=-1, keepdims=True)
        var = jnp.mean((y - mu) ** 2, axis=-1, keepdims=True)
        return lng[None, :] * (y - mu) * lax.rsqrt(var + 1e-6) + lnb[None, :]

    def body(q_ref, sv_ref, p1w_ref, p1b_ref, p2w_ref, p2b_ref, lng_ref,
             lnb_ref, wih_ref, whh_ref, bih_ref, bhh_ref, out_ref):
        p1w, p1b = p1w_ref[...], p1b_ref[...]
        p2w, p2b = p2w_ref[...], p2b_ref[...]
        lng, lnb = lng_ref[...], lnb_ref[...]
        wih, whh = wih_ref[...], whh_ref[...]
        bias = (bih_ref[...] + bhh_ref[...])[None, :]

        sg = jnp.mean(enc(sv_ref[...], p1w, p1b, p2w, p2b, lng, lnb),
                      axis=0, keepdims=True)
        qe = enc(q_ref[...], p1w, p1b, p2w, p2b, lng, lnb)

        qc = _mm_t(qe, wih) + bias
        whh_l = whh[:, :d_model]
        whh_r = whh[:, d_model:]
        rcon = _mm_t(sg, whh_r)

        hid = 2 * d_model
        c = jnp.zeros((blk, hid), jnp.float32)
        h = qe
        for step in range(4):
            if step == 0:
                gates = qc
            else:
                gates = qc + _mm_t(h, whh_l) + rcon
            gi = gates[:, 0 * hid:1 * hid]
            gf = gates[:, 1 * hid:2 * hid]
            gg = gates[:, 2 * hid:3 * hid]
            go = gates[:, 3 * hid:4 * hid]
            c = jax.nn.sigmoid(gf) * c + jax.nn.sigmoid(gi) * jnp.tanh(gg)
            h_r = jax.nn.sigmoid(go) * jnp.tanh(c)
            h = qe + h_r[:, :d_model]
        out_ref[...] = jnp.sum(h * sg, axis=1)

    return pl.pallas_call(
        body,
        grid=(grid,),
        in_specs=[
            pl.BlockSpec((blk, d_model), lambda g: (g, 0)),
            pl.BlockSpec((B_S, d_model), lambda g: (0, 0)),
            pl.BlockSpec((2 * d_model, d_model), lambda g: (0, 0)),
            pl.BlockSpec((2 * d_model,), lambda g: (0,)),
            pl.BlockSpec((d_model, 2 * d_model), lambda g: (0, 0)),
            pl.BlockSpec((d_model,), lambda g: (0,)),
            pl.BlockSpec((d_model,), lambda g: (0,)),
            pl.BlockSpec((d_model,), lambda g: (0,)),
            pl.BlockSpec((8 * d_model, d_model), lambda g: (0, 0)),
            pl.BlockSpec((8 * d_model, 2 * d_model), lambda g: (0, 0)),
            pl.BlockSpec((8 * d_model,), lambda g: (0,)),
            pl.BlockSpec((8 * d_model,), lambda g: (0,)),
        ],
        out_specs=pl.BlockSpec((blk,), lambda g: (g,)),
        out_shape=jax.ShapeDtypeStruct((B_Q,), jnp.float32),
    )(query_vec, support_vec, p1w, p1b, p2w, p2b, lng, lnb, wih, whh, bih,
      bhh)


def kernel(symbol_emb, gcn_w, gcn_wb, gcn_b, gate_w, gate_wb, gate_b, p1w,
           p1b, p2w, p2b, lng, lnb, wih, whh, bih, bhh, query, support, q_l1,
           q_e2, q_deg_l, q_r1, q_e5, q_deg_r, s_l1, s_e2, s_deg_l, s_r1,
           s_e5, s_deg_r):
    npad = NB - NROWS
    conn = jnp.concatenate([q_l1, q_r1, s_l1, s_r1], axis=0)
    conn_fill = (jnp.arange(npad * KMAX * 2, dtype=jnp.int32) % NUM_SYM
                 ).reshape(npad, KMAX, 2)
    conn = jnp.concatenate([conn, conn_fill], axis=0)
    selves = jnp.concatenate([query[:, 0], query[:, 1], support[:, 0],
                              support[:, 1],
                              jnp.arange(npad, dtype=jnp.int32) % NUM_SYM])
    rel_ids = conn[:, :, 0]
    ent_ids = conn[:, :, 1]

    nsub = 3
    sb = NB // nsub
    rel_parts, ent_parts, self_parts = [], [], []
    for s in range(nsub):
        o = s * sb
        ids3 = lax.dynamic_slice_in_dim(ent_ids, o, sb).reshape(
            NW, 2 * (sb // NW), HALF)
        selfids2 = lax.dynamic_slice_in_dim(selves, o, sb).reshape(
            NW, sb // NW)
        parts, selfrows = _sc_sims(symbol_emb, ids3, selfids2, nb=sb)
        dot2d, esq2d = _tc_groupsum(parts, blk=128, nb=sb)
        rel_sel, ent_sel = _tc_simtopk(
            dot2d, esq2d, selfrows,
            lax.dynamic_slice_in_dim(rel_ids, o, sb),
            lax.dynamic_slice_in_dim(ent_ids, o, sb), blk=128, nb=sb)
        rel_parts.append(rel_sel)
        ent_parts.append(ent_sel)
        self_parts.append(selfrows)
    rel_sel = jnp.concatenate(rel_parts, axis=0)
    ent_sel = jnp.concatenate(ent_parts, axis=0)
    selfrows = jnp.concatenate(self_parts, axis=0)

    ids_c = jnp.stack([rel_sel, ent_sel], axis=-1).reshape(-1)
    pad_c = NW * 12 * 128 - ids_c.shape[0]
    ids_c = jnp.concatenate(
        [ids_c, jnp.arange(pad_c, dtype=jnp.int32) % NUM_SYM]
    ).reshape(NW, 1, 12 * 128)
    rows_c = _sc_gather(symbol_emb, ids_c, nch=12, chunk=128)

    nbout = _tc_neighbor(rows_c, selfrows, gcn_w, gcn_wb, gcn_b, gate_w,
                         gate_wb, gate_b, blk=128, nb=NB)

    query_vec = jnp.concatenate([nbout[:B_Q], nbout[B_Q:2 * B_Q]], axis=1)
    support_vec = jnp.concatenate(
        [nbout[2 * B_Q:2 * B_Q + B_S], nbout[2 * B_Q + B_S:NROWS]], axis=1)

    return _tc_tail(query_vec, support_vec, p1w, p1b, p2w, p2b, lng, lnb,
                    wih, whh, bih, bhh, blk=256)
